# baseline (speedup 1.0000x reference)
import functools

import jax
import jax.numpy as jnp
from jax import lax
from jax.experimental import pallas as pl
from jax.experimental.pallas import tpu as pltpu
from jax.experimental.pallas import tpu_sc as plsc

N = 25000
D = 64
E = 800000
CK = 80
NCH = E // CK
NCHP = 10240
SUP = 10
GRP = 20
WCH = NCHP // 32
NP = 25088
STRIPE = NP // 16
ROWS = 1560
R = 5000
GRID = N // R

_MESH = dict(core_axis_name="c", subcore_axis_name="s", num_cores=2,
             num_subcores=16)

_f32 = jnp.float32
_i32 = jnp.int32


def _fast_rsqrt(y):
    i = lax.bitcast_convert_type(y, _i32)
    i = jnp.int32(0x5F3759DF) - lax.shift_right_logical(i, 1)
    r = lax.bitcast_convert_type(i, _f32)
    for _ in range(3):
        r = r * (1.5 - 0.5 * y * r * r)
    return r


def _weights_body(sd_c, ea_c, sd_v, ea_v,
                  w_c, w_v,
                  deg_s_sh, deg_d_sh, rsq_s_sh, rsq_d_sh,
                  sdS, eaS, wS, ones, stripe, rsql_s, rsql_d, ssem):
    c = lax.axis_index("c")
    s = lax.axis_index("s")

    for g in range(CK // 16):
        ones[pl.ds(g * 16, 16)] = jnp.ones((16,), _f32)

    def zero_stripe(v, _):
        stripe[pl.ds(v * 16, 16)] = jnp.zeros((16,), _f32)
        return 0
    lax.fori_loop(0, STRIPE // 16, zero_stripe, 0)
    pltpu.sync_copy(stripe, deg_s_sh.at[pl.ds(s * STRIPE, STRIPE)])
    pltpu.sync_copy(stripe, deg_d_sh.at[pl.ds(s * STRIPE, STRIPE)])
    plsc.subcore_barrier()

    per_tile = NCHP // 16
    start = s * per_tile

    def deg_pass(sd):
        def sbody(sp, _):
            base = start + sp * SUP

            @pl.when(base < NCH)
            def _():
                pltpu.sync_copy(sd.at[pl.ds(base, SUP)], sdS)
                descs = []
                for j in range(SUP):
                    descs.append(pltpu.async_copy(
                        ones, deg_s_sh.at[sdS.at[j, 0]], ssem, add=True))
                    descs.append(pltpu.async_copy(
                        ones, deg_d_sh.at[sdS.at[j, 1]], ssem, add=True))
                for d in descs:
                    d.wait()
            return 0
        lax.fori_loop(0, per_tile // SUP, sbody, 0)

    @pl.when(c == 0)
    def _():
        deg_pass(sd_c)

    @pl.when(c == 1)
    def _():
        deg_pass(sd_v)

    plsc.subcore_barrier()

    def rsqrt_stripe(deg_sh, rsq_sh):
        pltpu.sync_copy(deg_sh.at[pl.ds(s * STRIPE, STRIPE)], stripe)

        def body(v, _):
            y = jnp.maximum(stripe[pl.ds(v * 16, 16)], 1.0)
            stripe[pl.ds(v * 16, 16)] = _fast_rsqrt(y)
            return 0
        lax.fori_loop(0, STRIPE // 16, body, 0)
        pltpu.sync_copy(stripe, rsq_sh.at[pl.ds(s * STRIPE, STRIPE)])

    rsqrt_stripe(deg_s_sh, rsq_s_sh)
    rsqrt_stripe(deg_d_sh, rsq_d_sh)
    plsc.subcore_barrier()

    pltpu.sync_copy(rsq_s_sh, rsql_s)
    pltpu.sync_copy(rsq_d_sh, rsql_d)

    def w_pass(sd, ea, wout):
        def sbody(sp, _):
            base = start + sp * SUP
            pltpu.sync_copy(sd.at[pl.ds(base, SUP)], sdS)
            pltpu.sync_copy(ea.at[pl.ds(base, SUP)], eaS)
            for j in range(SUP):
                for g in range(CK // 16):
                    si = sdS[j, 0, pl.ds(g * 16, 16)]
                    di = sdS[j, 1, pl.ds(g * 16, 16)]
                    gs = plsc.load_gather(rsql_s, [si])
                    gd = plsc.load_gather(rsql_d, [di])
                    wS[j, pl.ds(g * 16, 16)] = (
                        eaS[j, pl.ds(g * 16, 16)] * gs * gd)
            pltpu.sync_copy(wS, wout.at[pl.ds(base, SUP)])
            return 0
        lax.fori_loop(0, per_tile // SUP, sbody, 0)

    @pl.when(c == 0)
    def _():
        w_pass(sd_c, ea_c, w_c)

    @pl.when(c == 1)
    def _():
        w_pass(sd_v, ea_v, w_v)


def _weights(sd_c, ea_c, sd_v, ea_v):
    k = pl.kernel(
        _weights_body,
        out_type=[jax.ShapeDtypeStruct((NCHP, CK), _f32),
                  jax.ShapeDtypeStruct((NCHP, CK), _f32)],
        mesh=plsc.VectorSubcoreMesh(**_MESH),
        compiler_params=pltpu.CompilerParams(needs_layout_passes=False, use_tc_tiling_on_sc=False),
        scratch_types=[
            pltpu.VMEM_SHARED((NP,), _f32),
            pltpu.VMEM_SHARED((NP,), _f32),
            pltpu.VMEM_SHARED((NP,), _f32),
            pltpu.VMEM_SHARED((NP,), _f32),
            pltpu.VMEM((SUP, 2, CK), _i32),
            pltpu.VMEM((SUP, CK), _f32),
            pltpu.VMEM((SUP, CK), _f32),
            pltpu.VMEM((CK,), _f32),
            pltpu.VMEM((STRIPE,), _f32),
            pltpu.VMEM((NP,), _f32),
            pltpu.VMEM((NP,), _f32),
            pltpu.SemaphoreType.DMA,
        ],
    )
    return k(sd_c, ea_c, sd_v, ea_v)


_ZFULL = ROWS // CK
_ZREM = ROWS - _ZFULL * CK
_TAIL = N - 16 * ROWS


def _spmm_body(xc_hbm, sd_c, w_c, xv_hbm, sd_v, w_v,
               outv_hbm, outc_hbm,
               acc, sdA, sdB, wwA, wwB, r0, r1, r2, r3,
               g0, g1, g2, g3, s0, s1, s2, s3, isemA, isemB):
    rows = (r0, r1, r2, r3)
    gsem = (g0, g1, g2, g3)
    ssem = (s0, s1, s2, s3)
    c = lax.axis_index("c")
    s = lax.axis_index("s")

    @plsc.parallel_loop(0, CK)
    def _(rr):
        for m in range(D // 16):
            r0[rr, pl.ds(m * 16, 16)] = jnp.zeros((16,), _f32)

    def zero_acc(k, _):
        pltpu.sync_copy(r0, acc.at[pl.ds(s * ROWS + k * CK, CK)])
        return 0
    lax.fori_loop(0, _ZFULL, zero_acc, 0)
    pltpu.sync_copy(r0.at[pl.ds(0, _ZREM)],
                    acc.at[pl.ds(s * ROWS + _ZFULL * CK, _ZREM)])

    @pl.when(s == 0)
    def _():
        pltpu.sync_copy(r0.at[pl.ds(0, _TAIL)],
                        acc.at[pl.ds(16 * ROWS, _TAIL)])

    plsc.subcore_barrier()

    def do_dir(x_hbm, sd_hbm, w_hbm, out_hbm):
        start = s * (NCHP // 16)

        def sref(k):
            return (sdA if k < SUP else sdB).at[k % SUP, 0]

        def dref(k):
            return (sdA if k < SUP else sdB).at[k % SUP, 1]

        def wref(k):
            return (wwA if k < SUP else wwB).at[k % SUP]

        def group(g, _):
            base = start + g * GRP
            dA = pltpu.async_copy(sd_hbm.at[pl.ds(base, SUP)], sdA, isemA)
            dAw = pltpu.async_copy(w_hbm.at[pl.ds(base, SUP)], wwA, isemA)
            dB = pltpu.async_copy(sd_hbm.at[pl.ds(base + SUP, SUP)], sdB,
                                  isemB)
            dBw = pltpu.async_copy(w_hbm.at[pl.ds(base + SUP, SUP)], wwB,
                                   isemB)
            dA.wait()
            dAw.wait()
            gd = {}
            sc = {}
            gd[0] = pltpu.async_copy(x_hbm.at[sref(0)], rows[0], gsem[0])
            gd[1] = pltpu.async_copy(x_hbm.at[sref(1)], rows[1], gsem[1])
            for k in range(2, GRP + 2):
                if k == SUP:
                    dB.wait()
                    dBw.wait()
                j = k - 2
                gd[j].wait()
                rr = rows[j % 4]
                wr = wref(j)

                @plsc.parallel_loop(0, CK, unroll=8)
                def _(e):
                    wb = plsc.load_gather(wr, [lax.broadcast(e, (16,))])
                    for m in range(D // 16):
                        rr[e, pl.ds(m * 16, 16)] = (
                            rr[e, pl.ds(m * 16, 16)] * wb)

                sc[j] = pltpu.async_copy(rr, acc.at[dref(j)], ssem[j % 4],
                                         add=True)
                if k < GRP:
                    if k >= 4:
                        sc[k - 4].wait()
                    gd[k] = pltpu.async_copy(x_hbm.at[sref(k)], rows[k % 4],
                                             gsem[k % 4])
            for j in range(GRP - 4, GRP):
                sc[j].wait()
            return 0

        lax.fori_loop(0, NCHP // 16 // GRP, group, 0)
        plsc.subcore_barrier()

        def out_copy(k, _):
            base = s * ROWS + k * CK
            pltpu.sync_copy(acc.at[pl.ds(base, CK)], r0)
            pltpu.sync_copy(r0, out_hbm.at[pl.ds(base, CK)])
            return 0
        lax.fori_loop(0, _ZFULL, out_copy, 0)
        rem_base = s * ROWS + _ZFULL * CK
        pltpu.sync_copy(acc.at[pl.ds(rem_base, _ZREM)],
                        r0.at[pl.ds(0, _ZREM)])
        pltpu.sync_copy(r0.at[pl.ds(0, _ZREM)],
                        out_hbm.at[pl.ds(rem_base, _ZREM)])

        @pl.when(s == 0)
        def _():
            pltpu.sync_copy(acc.at[pl.ds(16 * ROWS, _TAIL)],
                            r0.at[pl.ds(0, _TAIL)])
            pltpu.sync_copy(r0.at[pl.ds(0, _TAIL)],
                            out_hbm.at[pl.ds(16 * ROWS, _TAIL)])

    @pl.when(c == 0)
    def _():
        do_dir(xc_hbm, sd_c, w_c, outv_hbm)

    @pl.when(c == 1)
    def _():
        do_dir(xv_hbm, sd_v, w_v, outc_hbm)


def _spmm(xc, sd_c, w_c, xv, sd_v, w_v):
    k = pl.kernel(
        _spmm_body,
        out_type=[jax.ShapeDtypeStruct((N, D), _f32),
                  jax.ShapeDtypeStruct((N, D), _f32)],
        mesh=plsc.VectorSubcoreMesh(**_MESH),
        compiler_params=pltpu.CompilerParams(needs_layout_passes=False, use_tc_tiling_on_sc=False),
        scratch_types=(
            [pltpu.VMEM_SHARED((N, D), _f32)]
            + [pltpu.VMEM((SUP, 2, CK), _i32)] * 2
            + [pltpu.VMEM((SUP, CK), _f32)] * 2
            + [pltpu.VMEM((CK, D), _f32)] * 4
            + [pltpu.SemaphoreType.DMA] * 10
        ),
    )
    return k(xc, sd_c, w_c, xv, sd_v, w_v)


def _enc1_body(x_ref, w1_ref, b1_ref, h_ref, st_ref):
    i = pl.program_id(0)
    h = jnp.dot(x_ref[...], w1_ref[...], preferred_element_type=_f32)
    h = h + b1_ref[...]
    h_ref[...] = h
    st = jnp.concatenate(
        [jnp.sum(h, axis=0, keepdims=True),
         jnp.sum(h * h, axis=0, keepdims=True),
         jnp.zeros((6, D), _f32)], axis=0)

    @pl.when(i == 0)
    def _():
        st_ref[...] = st

    @pl.when(i > 0)
    def _():
        st_ref[...] = st_ref[...] + st


def _enc2_body(h_ref, st_ref, g1_ref, be1_ref, w2_ref, b2_ref, o_ref):
    st = st_ref[...]
    mu = st[0:1] * (1.0 / N)
    var = st[1:2] * (1.0 / N) - mu * mu
    xn = (h_ref[...] - mu) * lax.rsqrt(var + 1e-5) * g1_ref[...] + be1_ref[...]
    xn = jnp.maximum(xn, 0.0)
    o_ref[...] = jnp.dot(xn, w2_ref[...],
                         preferred_element_type=_f32) + b2_ref[...]


def _encode(x, p):
    h, st = pl.pallas_call(
        _enc1_body,
        grid=(GRID,),
        in_specs=[pl.BlockSpec((R, D), lambda i: (i, 0)),
                  pl.BlockSpec((D, D), lambda i: (0, 0)),
                  pl.BlockSpec((1, D), lambda i: (0, 0))],
        out_specs=[pl.BlockSpec((R, D), lambda i: (i, 0)),
                   pl.BlockSpec((8, D), lambda i: (0, 0))],
        out_shape=[jax.ShapeDtypeStruct((N, D), _f32),
                   jax.ShapeDtypeStruct((8, D), _f32)],
    )(x, p['W1'], p['b1'].reshape(1, D))
    return pl.pallas_call(
        _enc2_body,
        grid=(GRID,),
        in_specs=[pl.BlockSpec((R, D), lambda i: (i, 0)),
                  pl.BlockSpec((8, D), lambda i: (0, 0)),
                  pl.BlockSpec((1, D), lambda i: (0, 0)),
                  pl.BlockSpec((1, D), lambda i: (0, 0)),
                  pl.BlockSpec((D, D), lambda i: (0, 0)),
                  pl.BlockSpec((1, D), lambda i: (0, 0))],
        out_specs=pl.BlockSpec((R, D), lambda i: (i, 0)),
        out_shape=jax.ShapeDtypeStruct((N, D), _f32),
    )(h, st, p['g1'].reshape(1, D), p['be1'].reshape(1, D),
      p['W2'], p['b2'].reshape(1, D))


def _layer_body(agg_ref, xold_ref, w1_ref, b1_ref, w2_ref, b2_ref,
                h2_ref, xnew_ref):
    agg = agg_ref[...]
    h = jnp.maximum(
        jnp.dot(agg, w1_ref[...], preferred_element_type=_f32) + b1_ref[...],
        0.0)
    h2 = jnp.dot(h, w2_ref[...], preferred_element_type=_f32) + b2_ref[...]
    h2_ref[...] = h2
    xnew_ref[...] = (jnp.maximum(h2, 0.0) + xold_ref[...]) * 0.5


def _layer(agg2, xold, p):
    return pl.pallas_call(
        _layer_body,
        grid=(GRID,),
        in_specs=[pl.BlockSpec((R, D), lambda i: (i, 0)),
                  pl.BlockSpec((R, D), lambda i: (i, 0)),
                  pl.BlockSpec((D, D), lambda i: (0, 0)),
                  pl.BlockSpec((1, D), lambda i: (0, 0)),
                  pl.BlockSpec((D, D), lambda i: (0, 0)),
                  pl.BlockSpec((1, D), lambda i: (0, 0))],
        out_specs=[pl.BlockSpec((R, D), lambda i: (i, 0)),
                   pl.BlockSpec((R, D), lambda i: (i, 0))],
        out_shape=[jax.ShapeDtypeStruct((N, D), _f32),
                   jax.ShapeDtypeStruct((N, D), _f32)],
    )(agg2, xold, p['W1'], p['b1'].reshape(1, D),
      p['W2'], p['b2'].reshape(1, D))


def _pred_body(h0_ref, h1_ref, h2_ref, w1_ref, b1_ref, w2_ref, b2_ref,
               o_ref):
    cols = []
    for hr in (h0_ref, h1_ref, h2_ref):
        t = jnp.maximum(
            jnp.dot(hr[...], w1_ref[...], preferred_element_type=_f32)
            + b1_ref[...], 0.0)
        cols.append(jnp.dot(t, w2_ref[...],
                            preferred_element_type=_f32) + b2_ref[...])
    o_ref[...] = jnp.concatenate(cols, axis=1)


def _pred(hs, p):
    return pl.pallas_call(
        _pred_body,
        grid=(GRID,),
        in_specs=[pl.BlockSpec((R, D), lambda i: (i, 0)),
                  pl.BlockSpec((R, D), lambda i: (i, 0)),
                  pl.BlockSpec((R, D), lambda i: (i, 0)),
                  pl.BlockSpec((D, D), lambda i: (0, 0)),
                  pl.BlockSpec((1, D), lambda i: (0, 0)),
                  pl.BlockSpec((D, 1), lambda i: (0, 0)),
                  pl.BlockSpec((1, 1), lambda i: (0, 0))],
        out_specs=pl.BlockSpec((R, 3), lambda i: (i, 0)),
        out_shape=jax.ShapeDtypeStruct((N, 3), _f32),
    )(hs[0], hs[1], hs[2], p['W1'], p['b1'].reshape(1, D),
      p['W2'], p['b2'].reshape(1, 1))


def kernel(x_cons, x_vals, edge_index_c2v, edge_index_v2c,
           edge_attr_c2v, edge_attr_v2c, params):
    pr = params
    padE = NCHP * CK - E
    fill = jnp.arange(padE, dtype=_i32) % N

    def prep(ei, ea):
        src = jnp.concatenate([ei[0].astype(_i32), fill]).reshape(NCHP, CK)
        dst = jnp.concatenate([ei[1].astype(_i32), fill]).reshape(NCHP, CK)
        eap = jnp.concatenate(
            [ea.astype(_f32).reshape(E), jnp.zeros((padE,), _f32)]
        ).reshape(NCHP, CK)
        return jnp.stack([src, dst], axis=1), eap

    sd_c, ea_c = prep(edge_index_c2v, edge_attr_c2v)
    sd_v, ea_v = prep(edge_index_v2c, edge_attr_v2c)
    w_c, w_v = _weights(sd_c, ea_c, sd_v, ea_v)
    xc = _encode(x_cons, pr['enc_cons'])
    xv = _encode(x_vals, pr['enc_vals'])

    hv, hc = [], []
    for i in range(3):
        aggv, aggc = _spmm(xc, sd_c, w_c, xv, sd_v, w_v)
        h2v, xv = _layer(aggv, xv, pr['convs'][i]['c2v'])
        h2c, xc = _layer(aggc, xc, pr['convs'][i]['v2c'])
        hv.append(h2v)
        hc.append(h2c)

    vals = _pred(hv, pr['pred_vals'])
    cons = _pred(hc, pr['pred_cons'])
    return (vals, cons)


# speedup vs baseline: 1.0458x; 1.0458x over previous
import functools

import jax
import jax.numpy as jnp
from jax import lax
from jax.experimental import pallas as pl
from jax.experimental.pallas import tpu as pltpu
from jax.experimental.pallas import tpu_sc as plsc

N = 25000
D = 64
E = 800000
CK = 80
NCH = E // CK
NCHP = 10240
SUP = 10
GRP = 20
WCH = NCHP // 32
NP = 25088
STRIPE = NP // 16
ROWS = 1560
R = 5000
GRID = N // R

_MESH = dict(core_axis_name="c", subcore_axis_name="s", num_cores=2,
             num_subcores=16)

_f32 = jnp.float32
_i32 = jnp.int32


def _fast_rsqrt(y):
    i = lax.bitcast_convert_type(y, _i32)
    i = jnp.int32(0x5F3759DF) - lax.shift_right_logical(i, 1)
    r = lax.bitcast_convert_type(i, _f32)
    for _ in range(3):
        r = r * (1.5 - 0.5 * y * r * r)
    return r


def _weights_body(sd_c, ea_c, sd_v, ea_v,
                  w_c, w_v,
                  deg_s_sh, deg_d_sh, rsq_s_sh, rsq_d_sh,
                  sdS, eaS, wS, ones, stripe, rsql_s, rsql_d, ssem):
    c = lax.axis_index("c")
    s = lax.axis_index("s")

    for g in range(CK // 16):
        ones[pl.ds(g * 16, 16)] = jnp.ones((16,), _f32)

    def zero_stripe(v, _):
        stripe[pl.ds(v * 16, 16)] = jnp.zeros((16,), _f32)
        return 0
    lax.fori_loop(0, STRIPE // 16, zero_stripe, 0)
    pltpu.sync_copy(stripe, deg_s_sh.at[pl.ds(s * STRIPE, STRIPE)])
    pltpu.sync_copy(stripe, deg_d_sh.at[pl.ds(s * STRIPE, STRIPE)])
    plsc.subcore_barrier()

    per_tile = NCHP // 16
    start = s * per_tile

    def deg_pass(sd):
        def sbody(sp, _):
            base = start + sp * SUP

            @pl.when(base < NCH)
            def _():
                pltpu.sync_copy(sd.at[pl.ds(base, SUP)], sdS)
                descs = []
                for j in range(SUP):
                    descs.append(pltpu.async_copy(
                        ones, deg_s_sh.at[sdS.at[j, 0]], ssem, add=True))
                    descs.append(pltpu.async_copy(
                        ones, deg_d_sh.at[sdS.at[j, 1]], ssem, add=True))
                for d in descs:
                    d.wait()
            return 0
        lax.fori_loop(0, per_tile // SUP, sbody, 0)

    @pl.when(c == 0)
    def _():
        deg_pass(sd_c)

    @pl.when(c == 1)
    def _():
        deg_pass(sd_v)

    plsc.subcore_barrier()

    def rsqrt_stripe(deg_sh, rsq_sh):
        pltpu.sync_copy(deg_sh.at[pl.ds(s * STRIPE, STRIPE)], stripe)

        def body(v, _):
            y = jnp.maximum(stripe[pl.ds(v * 16, 16)], 1.0)
            stripe[pl.ds(v * 16, 16)] = _fast_rsqrt(y)
            return 0
        lax.fori_loop(0, STRIPE // 16, body, 0)
        pltpu.sync_copy(stripe, rsq_sh.at[pl.ds(s * STRIPE, STRIPE)])

    rsqrt_stripe(deg_s_sh, rsq_s_sh)
    rsqrt_stripe(deg_d_sh, rsq_d_sh)
    plsc.subcore_barrier()

    pltpu.sync_copy(rsq_s_sh, rsql_s)
    pltpu.sync_copy(rsq_d_sh, rsql_d)

    def w_pass(sd, ea, wout):
        def sbody(sp, _):
            base = start + sp * SUP
            pltpu.sync_copy(sd.at[pl.ds(base, SUP)], sdS)
            pltpu.sync_copy(ea.at[pl.ds(base, SUP)], eaS)
            for j in range(SUP):
                for g in range(CK // 16):
                    si = sdS[j, 0, pl.ds(g * 16, 16)]
                    di = sdS[j, 1, pl.ds(g * 16, 16)]
                    gs = plsc.load_gather(rsql_s, [si])
                    gd = plsc.load_gather(rsql_d, [di])
                    wS[j, pl.ds(g * 16, 16)] = (
                        eaS[j, pl.ds(g * 16, 16)] * gs * gd)
            pltpu.sync_copy(wS, wout.at[pl.ds(base, SUP)])
            return 0
        lax.fori_loop(0, per_tile // SUP, sbody, 0)

    @pl.when(c == 0)
    def _():
        w_pass(sd_c, ea_c, w_c)

    @pl.when(c == 1)
    def _():
        w_pass(sd_v, ea_v, w_v)


def _weights(sd_c, ea_c, sd_v, ea_v):
    k = pl.kernel(
        _weights_body,
        out_type=[jax.ShapeDtypeStruct((NCHP, CK), _f32),
                  jax.ShapeDtypeStruct((NCHP, CK), _f32)],
        mesh=plsc.VectorSubcoreMesh(**_MESH),
        compiler_params=pltpu.CompilerParams(needs_layout_passes=False, use_tc_tiling_on_sc=False),
        scratch_types=[
            pltpu.VMEM_SHARED((NP,), _f32),
            pltpu.VMEM_SHARED((NP,), _f32),
            pltpu.VMEM_SHARED((NP,), _f32),
            pltpu.VMEM_SHARED((NP,), _f32),
            pltpu.VMEM((SUP, 2, CK), _i32),
            pltpu.VMEM((SUP, CK), _f32),
            pltpu.VMEM((SUP, CK), _f32),
            pltpu.VMEM((CK,), _f32),
            pltpu.VMEM((STRIPE,), _f32),
            pltpu.VMEM((NP,), _f32),
            pltpu.VMEM((NP,), _f32),
            pltpu.SemaphoreType.DMA,
        ],
    )
    return k(sd_c, ea_c, sd_v, ea_v)


_ZFULL = ROWS // CK
_ZREM = ROWS - _ZFULL * CK
_TAIL = N - 16 * ROWS


def _spmm_body(x_hbm, sd_hbm, w_hbm,
               out_hbm,
               acc, sdA, sdB, wwA, wwB, r0, r1, r2, r3,
               g0, g1, g2, g3, s0, s1, s2, s3, isemA, isemB):
    rows = (r0, r1, r2, r3)
    gsem = (g0, g1, g2, g3)
    ssem = (s0, s1, s2, s3)
    c = lax.axis_index("c")
    s = lax.axis_index("s")

    @plsc.parallel_loop(0, CK)
    def _(rr):
        for m in range(D // 16):
            r0[rr, pl.ds(m * 16, 16)] = jnp.zeros((16,), _f32)

    def zero_acc(k, _):
        pltpu.sync_copy(r0, acc.at[pl.ds(s * ROWS + k * CK, CK)])
        return 0
    lax.fori_loop(0, _ZFULL, zero_acc, 0)
    pltpu.sync_copy(r0.at[pl.ds(0, _ZREM)],
                    acc.at[pl.ds(s * ROWS + _ZFULL * CK, _ZREM)])

    @pl.when(s == 0)
    def _():
        pltpu.sync_copy(r0.at[pl.ds(0, _TAIL)],
                        acc.at[pl.ds(16 * ROWS, _TAIL)])

    plsc.subcore_barrier()

    start = (c * 16 + s) * WCH

    def sref(k):
        return (sdA if k < SUP else sdB).at[k % SUP, 0]

    def dref(k):
        return (sdA if k < SUP else sdB).at[k % SUP, 1]

    def wref(k):
        return (wwA if k < SUP else wwB).at[k % SUP]

    def group(g, _):
        base = start + g * GRP
        dA = pltpu.async_copy(sd_hbm.at[pl.ds(base, SUP)], sdA, isemA)
        dAw = pltpu.async_copy(w_hbm.at[pl.ds(base, SUP)], wwA, isemA)
        dB = pltpu.async_copy(sd_hbm.at[pl.ds(base + SUP, SUP)], sdB, isemB)
        dBw = pltpu.async_copy(w_hbm.at[pl.ds(base + SUP, SUP)], wwB, isemB)
        dA.wait()
        dAw.wait()
        gd = {}
        sc = {}
        gd[0] = pltpu.async_copy(x_hbm.at[sref(0)], rows[0], gsem[0])
        gd[1] = pltpu.async_copy(x_hbm.at[sref(1)], rows[1], gsem[1])
        for k in range(2, GRP + 2):
            if k == SUP:
                dB.wait()
                dBw.wait()
            j = k - 2
            gd[j].wait()
            rr = rows[j % 4]
            wr = wref(j)

            @plsc.parallel_loop(0, CK, unroll=8)
            def _(e):
                wb = plsc.load_gather(wr, [lax.broadcast(e, (16,))])
                for m in range(D // 16):
                    rr[e, pl.ds(m * 16, 16)] = rr[e, pl.ds(m * 16, 16)] * wb

            sc[j] = pltpu.async_copy(rr, acc.at[dref(j)], ssem[j % 4],
                                     add=True)
            if k < GRP:
                if k >= 4:
                    sc[k - 4].wait()
                gd[k] = pltpu.async_copy(x_hbm.at[sref(k)], rows[k % 4],
                                         gsem[k % 4])
        for j in range(GRP - 4, GRP):
            sc[j].wait()
        return 0

    lax.fori_loop(0, WCH // GRP, group, 0)
    plsc.subcore_barrier()

    def out_copy(k, _):
        base = s * ROWS + k * CK
        pltpu.sync_copy(acc.at[pl.ds(base, CK)], r0)
        pltpu.sync_copy(r0, out_hbm.at[c, pl.ds(base, CK)])
        return 0
    lax.fori_loop(0, _ZFULL, out_copy, 0)
    rem_base = s * ROWS + _ZFULL * CK
    pltpu.sync_copy(acc.at[pl.ds(rem_base, _ZREM)], r0.at[pl.ds(0, _ZREM)])
    pltpu.sync_copy(r0.at[pl.ds(0, _ZREM)],
                    out_hbm.at[c, pl.ds(rem_base, _ZREM)])

    @pl.when(s == 0)
    def _():
        pltpu.sync_copy(acc.at[pl.ds(16 * ROWS, _TAIL)],
                        r0.at[pl.ds(0, _TAIL)])
        pltpu.sync_copy(r0.at[pl.ds(0, _TAIL)],
                        out_hbm.at[c, pl.ds(16 * ROWS, _TAIL)])


def _spmm(x, sd, w):
    k = pl.kernel(
        _spmm_body,
        out_type=jax.ShapeDtypeStruct((2, N, D), _f32),
        mesh=plsc.VectorSubcoreMesh(**_MESH),
        compiler_params=pltpu.CompilerParams(needs_layout_passes=False, use_tc_tiling_on_sc=False),
        scratch_types=(
            [pltpu.VMEM_SHARED((N, D), _f32)]
            + [pltpu.VMEM((SUP, 2, CK), _i32)] * 2
            + [pltpu.VMEM((SUP, CK), _f32)] * 2
            + [pltpu.VMEM((CK, D), _f32)] * 4
            + [pltpu.SemaphoreType.DMA] * 10
        ),
    )
    return k(x, sd, w)


def _enc1_body(x_ref, w1_ref, b1_ref, h_ref, st_ref):
    i = pl.program_id(0)
    h = jnp.dot(x_ref[...], w1_ref[...], preferred_element_type=_f32)
    h = h + b1_ref[...]
    h_ref[...] = h
    st = jnp.concatenate(
        [jnp.sum(h, axis=0, keepdims=True),
         jnp.sum(h * h, axis=0, keepdims=True),
         jnp.zeros((6, D), _f32)], axis=0)

    @pl.when(i == 0)
    def _():
        st_ref[...] = st

    @pl.when(i > 0)
    def _():
        st_ref[...] = st_ref[...] + st


def _enc2_body(h_ref, st_ref, g1_ref, be1_ref, w2_ref, b2_ref, o_ref):
    st = st_ref[...]
    mu = st[0:1] * (1.0 / N)
    var = st[1:2] * (1.0 / N) - mu * mu
    xn = (h_ref[...] - mu) * lax.rsqrt(var + 1e-5) * g1_ref[...] + be1_ref[...]
    xn = jnp.maximum(xn, 0.0)
    o_ref[...] = jnp.dot(xn, w2_ref[...],
                         preferred_element_type=_f32) + b2_ref[...]


def _encode(x, p):
    h, st = pl.pallas_call(
        _enc1_body,
        grid=(GRID,),
        in_specs=[pl.BlockSpec((R, D), lambda i: (i, 0)),
                  pl.BlockSpec((D, D), lambda i: (0, 0)),
                  pl.BlockSpec((1, D), lambda i: (0, 0))],
        out_specs=[pl.BlockSpec((R, D), lambda i: (i, 0)),
                   pl.BlockSpec((8, D), lambda i: (0, 0))],
        out_shape=[jax.ShapeDtypeStruct((N, D), _f32),
                   jax.ShapeDtypeStruct((8, D), _f32)],
    )(x, p['W1'], p['b1'].reshape(1, D))
    return pl.pallas_call(
        _enc2_body,
        grid=(GRID,),
        in_specs=[pl.BlockSpec((R, D), lambda i: (i, 0)),
                  pl.BlockSpec((8, D), lambda i: (0, 0)),
                  pl.BlockSpec((1, D), lambda i: (0, 0)),
                  pl.BlockSpec((1, D), lambda i: (0, 0)),
                  pl.BlockSpec((D, D), lambda i: (0, 0)),
                  pl.BlockSpec((1, D), lambda i: (0, 0))],
        out_specs=pl.BlockSpec((R, D), lambda i: (i, 0)),
        out_shape=jax.ShapeDtypeStruct((N, D), _f32),
    )(h, st, p['g1'].reshape(1, D), p['be1'].reshape(1, D),
      p['W2'], p['b2'].reshape(1, D))


def _layer_body(agg_ref, xold_ref, w1_ref, b1_ref, w2_ref, b2_ref,
                pw1_ref, pb1_ref, pw2_ref, pb2_ref,
                xnew_ref, pcol_ref):
    agg = agg_ref[0] + agg_ref[1]
    h = jnp.maximum(
        jnp.dot(agg, w1_ref[...], preferred_element_type=_f32) + b1_ref[...],
        0.0)
    h2 = jnp.dot(h, w2_ref[...], preferred_element_type=_f32) + b2_ref[...]
    xnew_ref[...] = (jnp.maximum(h2, 0.0) + xold_ref[...]) * 0.5
    t = jnp.maximum(
        jnp.dot(h2, pw1_ref[...], preferred_element_type=_f32)
        + pb1_ref[...], 0.0)
    pcol_ref[...] = jnp.dot(t, pw2_ref[...],
                            preferred_element_type=_f32) + pb2_ref[...]


def _layer(agg2, xold, p, pp):
    return pl.pallas_call(
        _layer_body,
        grid=(GRID,),
        in_specs=[pl.BlockSpec((2, R, D), lambda i: (0, i, 0)),
                  pl.BlockSpec((R, D), lambda i: (i, 0)),
                  pl.BlockSpec((D, D), lambda i: (0, 0)),
                  pl.BlockSpec((1, D), lambda i: (0, 0)),
                  pl.BlockSpec((D, D), lambda i: (0, 0)),
                  pl.BlockSpec((1, D), lambda i: (0, 0)),
                  pl.BlockSpec((D, D), lambda i: (0, 0)),
                  pl.BlockSpec((1, D), lambda i: (0, 0)),
                  pl.BlockSpec((D, 1), lambda i: (0, 0)),
                  pl.BlockSpec((1, 1), lambda i: (0, 0))],
        out_specs=[pl.BlockSpec((R, D), lambda i: (i, 0)),
                   pl.BlockSpec((R, 1), lambda i: (i, 0))],
        out_shape=[jax.ShapeDtypeStruct((N, D), _f32),
                   jax.ShapeDtypeStruct((N, 1), _f32)],
    )(agg2, xold, p['W1'], p['b1'].reshape(1, D),
      p['W2'], p['b2'].reshape(1, D),
      pp['W1'], pp['b1'].reshape(1, D),
      pp['W2'], pp['b2'].reshape(1, 1))


def kernel(x_cons, x_vals, edge_index_c2v, edge_index_v2c,
           edge_attr_c2v, edge_attr_v2c, params):
    pr = params
    padE = NCHP * CK - E
    fill = jnp.arange(padE, dtype=_i32) % N

    def prep(ei, ea):
        src = jnp.concatenate([ei[0].astype(_i32), fill]).reshape(NCHP, CK)
        dst = jnp.concatenate([ei[1].astype(_i32), fill]).reshape(NCHP, CK)
        eap = jnp.concatenate(
            [ea.astype(_f32).reshape(E), jnp.zeros((padE,), _f32)]
        ).reshape(NCHP, CK)
        return jnp.stack([src, dst], axis=1), eap

    sd_c, ea_c = prep(edge_index_c2v, edge_attr_c2v)
    sd_v, ea_v = prep(edge_index_v2c, edge_attr_v2c)
    w_c, w_v = _weights(sd_c, ea_c, sd_v, ea_v)
    xc = _encode(x_cons, pr['enc_cons'])
    xv = _encode(x_vals, pr['enc_vals'])

    pv, pc = [], []
    for i in range(3):
        aggv = _spmm(xc, sd_c, w_c)
        aggc = _spmm(xv, sd_v, w_v)
        xv, pvi = _layer(aggv, xv, pr['convs'][i]['c2v'], pr['pred_vals'])
        xc, pci = _layer(aggc, xc, pr['convs'][i]['v2c'], pr['pred_cons'])
        pv.append(pvi)
        pc.append(pci)

    vals = jnp.concatenate(pv, axis=1)
    cons = jnp.concatenate(pc, axis=1)
    return (vals, cons)


# speedup vs baseline: 1.1964x; 1.1440x over previous
import functools

import jax
import jax.numpy as jnp
from jax import lax
from jax.experimental import pallas as pl
from jax.experimental.pallas import tpu as pltpu
from jax.experimental.pallas import tpu_sc as plsc

N = 25000
D = 64
E = 800000
CK = 128
NCH = E // CK
NCHP = 6400
SUP = 5
GRP = 10
WCH = NCHP // 32
NP = 25088
STRIPE = NP // 16
ROWS = 1560
R = 5000
GRID = N // R

_MESH = dict(core_axis_name="c", subcore_axis_name="s", num_cores=2,
             num_subcores=16)

_f32 = jnp.float32
_i32 = jnp.int32


def _fast_rsqrt(y):
    i = lax.bitcast_convert_type(y, _i32)
    i = jnp.int32(0x5F3759DF) - lax.shift_right_logical(i, 1)
    r = lax.bitcast_convert_type(i, _f32)
    for _ in range(3):
        r = r * (1.5 - 0.5 * y * r * r)
    return r


def _weights_body(sd_c, ea_c, sd_v, ea_v,
                  w_c, w_v,
                  deg_s_sh, deg_d_sh, rsq_s_sh, rsq_d_sh,
                  sdS, eaS, wS, ones, stripe, rsql_s, rsql_d, ssem):
    c = lax.axis_index("c")
    s = lax.axis_index("s")

    for g in range(CK // 16):
        ones[pl.ds(g * 16, 16)] = jnp.ones((16,), _f32)

    def zero_stripe(v, _):
        stripe[pl.ds(v * 16, 16)] = jnp.zeros((16,), _f32)
        return 0
    lax.fori_loop(0, STRIPE // 16, zero_stripe, 0)
    pltpu.sync_copy(stripe, deg_s_sh.at[pl.ds(s * STRIPE, STRIPE)])
    pltpu.sync_copy(stripe, deg_d_sh.at[pl.ds(s * STRIPE, STRIPE)])
    plsc.subcore_barrier()

    per_tile = NCHP // 16
    start = s * per_tile

    def deg_pass(sd):
        def sbody(sp, _):
            base = start + sp * SUP

            @pl.when(base < NCH)
            def _():
                pltpu.sync_copy(sd.at[pl.ds(base, SUP)], sdS)
                descs = []
                for j in range(SUP):
                    descs.append(pltpu.async_copy(
                        ones, deg_s_sh.at[sdS.at[j, 0]], ssem, add=True))
                    descs.append(pltpu.async_copy(
                        ones, deg_d_sh.at[sdS.at[j, 1]], ssem, add=True))
                for d in descs:
                    d.wait()
            return 0
        lax.fori_loop(0, per_tile // SUP, sbody, 0)

    @pl.when(c == 0)
    def _():
        deg_pass(sd_c)

    @pl.when(c == 1)
    def _():
        deg_pass(sd_v)

    plsc.subcore_barrier()

    def rsqrt_stripe(deg_sh, rsq_sh):
        pltpu.sync_copy(deg_sh.at[pl.ds(s * STRIPE, STRIPE)], stripe)

        def body(v, _):
            y = jnp.maximum(stripe[pl.ds(v * 16, 16)], 1.0)
            stripe[pl.ds(v * 16, 16)] = _fast_rsqrt(y)
            return 0
        lax.fori_loop(0, STRIPE // 16, body, 0)
        pltpu.sync_copy(stripe, rsq_sh.at[pl.ds(s * STRIPE, STRIPE)])

    rsqrt_stripe(deg_s_sh, rsq_s_sh)
    rsqrt_stripe(deg_d_sh, rsq_d_sh)
    plsc.subcore_barrier()

    pltpu.sync_copy(rsq_s_sh, rsql_s)
    pltpu.sync_copy(rsq_d_sh, rsql_d)

    def w_pass(sd, ea, wout):
        def sbody(sp, _):
            base = start + sp * SUP
            pltpu.sync_copy(sd.at[pl.ds(base, SUP)], sdS)
            pltpu.sync_copy(ea.at[pl.ds(base, SUP)], eaS)
            for j in range(SUP):
                for g in range(CK // 16):
                    si = sdS[j, 0, pl.ds(g * 16, 16)]
                    di = sdS[j, 1, pl.ds(g * 16, 16)]
                    gs = plsc.load_gather(rsql_s, [si])
                    gd = plsc.load_gather(rsql_d, [di])
                    wS[j, pl.ds(g * 16, 16)] = (
                        eaS[j, pl.ds(g * 16, 16)] * gs * gd)
            pltpu.sync_copy(wS, wout.at[pl.ds(base, SUP)])
            return 0
        lax.fori_loop(0, per_tile // SUP, sbody, 0)

    @pl.when(c == 0)
    def _():
        w_pass(sd_c, ea_c, w_c)

    @pl.when(c == 1)
    def _():
        w_pass(sd_v, ea_v, w_v)


def _weights(sd_c, ea_c, sd_v, ea_v):
    k = pl.kernel(
        _weights_body,
        out_type=[jax.ShapeDtypeStruct((NCHP, CK), _f32),
                  jax.ShapeDtypeStruct((NCHP, CK), _f32)],
        mesh=plsc.VectorSubcoreMesh(**_MESH),
        compiler_params=pltpu.CompilerParams(needs_layout_passes=False, use_tc_tiling_on_sc=False),
        scratch_types=[
            pltpu.VMEM_SHARED((NP,), _f32),
            pltpu.VMEM_SHARED((NP,), _f32),
            pltpu.VMEM_SHARED((NP,), _f32),
            pltpu.VMEM_SHARED((NP,), _f32),
            pltpu.VMEM((SUP, 2, CK), _i32),
            pltpu.VMEM((SUP, CK), _f32),
            pltpu.VMEM((SUP, CK), _f32),
            pltpu.VMEM((CK,), _f32),
            pltpu.VMEM((STRIPE,), _f32),
            pltpu.VMEM((NP,), _f32),
            pltpu.VMEM((NP,), _f32),
            pltpu.SemaphoreType.DMA,
        ],
    )
    return k(sd_c, ea_c, sd_v, ea_v)


_ZFULL = ROWS // CK
_ZREM = ROWS - _ZFULL * CK
_TAIL = N - 16 * ROWS


def _spmm_body(x_hbm, sd_hbm, w_hbm,
               out_hbm,
               acc, sdA, sdB, wwA, wwB, r0, r1, r2,
               g0, g1, g2, s0, s1, s2, isemA, isemB):
    rows = (r0, r1, r2)
    gsem = (g0, g1, g2)
    ssem = (s0, s1, s2)
    c = lax.axis_index("c")
    s = lax.axis_index("s")

    @plsc.parallel_loop(0, CK)
    def _(rr):
        for m in range(D // 16):
            r0[rr, pl.ds(m * 16, 16)] = jnp.zeros((16,), _f32)

    def zero_acc(k, _):
        pltpu.sync_copy(r0, acc.at[pl.ds(s * ROWS + k * CK, CK)])
        return 0
    lax.fori_loop(0, _ZFULL, zero_acc, 0)
    pltpu.sync_copy(r0.at[pl.ds(0, _ZREM)],
                    acc.at[pl.ds(s * ROWS + _ZFULL * CK, _ZREM)])

    @pl.when(s == 0)
    def _():
        pltpu.sync_copy(r0.at[pl.ds(0, _TAIL)],
                        acc.at[pl.ds(16 * ROWS, _TAIL)])

    plsc.subcore_barrier()

    start = (c * 16 + s) * WCH

    def sref(k):
        return (sdA if k < SUP else sdB).at[k % SUP, 0]

    def dref(k):
        return (sdA if k < SUP else sdB).at[k % SUP, 1]

    def wref(k):
        return (wwA if k < SUP else wwB).at[k % SUP]

    def group(g, _):
        base = start + g * GRP
        dA = pltpu.async_copy(sd_hbm.at[pl.ds(base, SUP)], sdA, isemA)
        dAw = pltpu.async_copy(w_hbm.at[pl.ds(base, SUP)], wwA, isemA)
        dB = pltpu.async_copy(sd_hbm.at[pl.ds(base + SUP, SUP)], sdB, isemB)
        dBw = pltpu.async_copy(w_hbm.at[pl.ds(base + SUP, SUP)], wwB, isemB)
        dA.wait()
        dAw.wait()
        gd = {}
        sc = {}
        gd[0] = pltpu.async_copy(x_hbm.at[sref(0)], rows[0], gsem[0])
        gd[1] = pltpu.async_copy(x_hbm.at[sref(1)], rows[1], gsem[1])
        for k in range(2, GRP + 2):
            if k == SUP:
                dB.wait()
                dBw.wait()
            j = k - 2
            gd[j].wait()
            rr = rows[j % 3]
            wr = wref(j)

            @plsc.parallel_loop(0, CK, unroll=8)
            def _(e):
                wb = plsc.load_gather(wr, [lax.broadcast(e, (16,))])
                for m in range(D // 16):
                    rr[e, pl.ds(m * 16, 16)] = rr[e, pl.ds(m * 16, 16)] * wb

            sc[j] = pltpu.async_copy(rr, acc.at[dref(j)], ssem[j % 3],
                                     add=True)
            if k < GRP:
                if k >= 3:
                    sc[k - 3].wait()
                gd[k] = pltpu.async_copy(x_hbm.at[sref(k)], rows[k % 3],
                                         gsem[k % 3])
        for j in range(GRP - 3, GRP):
            sc[j].wait()
        return 0

    lax.fori_loop(0, WCH // GRP, group, 0)
    plsc.subcore_barrier()

    def out_copy(k, _):
        base = s * ROWS + k * CK
        pltpu.sync_copy(acc.at[pl.ds(base, CK)], r0)
        pltpu.sync_copy(r0, out_hbm.at[c, pl.ds(base, CK)])
        return 0
    lax.fori_loop(0, _ZFULL, out_copy, 0)
    rem_base = s * ROWS + _ZFULL * CK
    pltpu.sync_copy(acc.at[pl.ds(rem_base, _ZREM)], r0.at[pl.ds(0, _ZREM)])
    pltpu.sync_copy(r0.at[pl.ds(0, _ZREM)],
                    out_hbm.at[c, pl.ds(rem_base, _ZREM)])

    @pl.when(s == 0)
    def _():
        pltpu.sync_copy(acc.at[pl.ds(16 * ROWS, _TAIL)],
                        r0.at[pl.ds(0, _TAIL)])
        pltpu.sync_copy(r0.at[pl.ds(0, _TAIL)],
                        out_hbm.at[c, pl.ds(16 * ROWS, _TAIL)])


def _spmm(x, sd, w):
    k = pl.kernel(
        _spmm_body,
        out_type=jax.ShapeDtypeStruct((2, N, D), _f32),
        mesh=plsc.VectorSubcoreMesh(**_MESH),
        compiler_params=pltpu.CompilerParams(needs_layout_passes=False, use_tc_tiling_on_sc=False),
        scratch_types=(
            [pltpu.VMEM_SHARED((N, D), _f32)]
            + [pltpu.VMEM((SUP, 2, CK), _i32)] * 2
            + [pltpu.VMEM((SUP, CK), _f32)] * 2
            + [pltpu.VMEM((CK, D), _f32)] * 3
            + [pltpu.SemaphoreType.DMA] * 8
        ),
    )
    return k(x, sd, w)


def _enc1_body(x_ref, w1_ref, b1_ref, h_ref, st_ref):
    i = pl.program_id(0)
    h = jnp.dot(x_ref[...], w1_ref[...], preferred_element_type=_f32)
    h = h + b1_ref[...]
    h_ref[...] = h
    st = jnp.concatenate(
        [jnp.sum(h, axis=0, keepdims=True),
         jnp.sum(h * h, axis=0, keepdims=True),
         jnp.zeros((6, D), _f32)], axis=0)

    @pl.when(i == 0)
    def _():
        st_ref[...] = st

    @pl.when(i > 0)
    def _():
        st_ref[...] = st_ref[...] + st


def _enc2_body(h_ref, st_ref, g1_ref, be1_ref, w2_ref, b2_ref, o_ref):
    st = st_ref[...]
    mu = st[0:1] * (1.0 / N)
    var = st[1:2] * (1.0 / N) - mu * mu
    xn = (h_ref[...] - mu) * lax.rsqrt(var + 1e-5) * g1_ref[...] + be1_ref[...]
    xn = jnp.maximum(xn, 0.0)
    o_ref[...] = jnp.dot(xn, w2_ref[...],
                         preferred_element_type=_f32) + b2_ref[...]


def _encode(x, p):
    h, st = pl.pallas_call(
        _enc1_body,
        grid=(GRID,),
        in_specs=[pl.BlockSpec((R, D), lambda i: (i, 0)),
                  pl.BlockSpec((D, D), lambda i: (0, 0)),
                  pl.BlockSpec((1, D), lambda i: (0, 0))],
        out_specs=[pl.BlockSpec((R, D), lambda i: (i, 0)),
                   pl.BlockSpec((8, D), lambda i: (0, 0))],
        out_shape=[jax.ShapeDtypeStruct((N, D), _f32),
                   jax.ShapeDtypeStruct((8, D), _f32)],
    )(x, p['W1'], p['b1'].reshape(1, D))
    return pl.pallas_call(
        _enc2_body,
        grid=(GRID,),
        in_specs=[pl.BlockSpec((R, D), lambda i: (i, 0)),
                  pl.BlockSpec((8, D), lambda i: (0, 0)),
                  pl.BlockSpec((1, D), lambda i: (0, 0)),
                  pl.BlockSpec((1, D), lambda i: (0, 0)),
                  pl.BlockSpec((D, D), lambda i: (0, 0)),
                  pl.BlockSpec((1, D), lambda i: (0, 0))],
        out_specs=pl.BlockSpec((R, D), lambda i: (i, 0)),
        out_shape=jax.ShapeDtypeStruct((N, D), _f32),
    )(h, st, p['g1'].reshape(1, D), p['be1'].reshape(1, D),
      p['W2'], p['b2'].reshape(1, D))


def _layer_body(agg_ref, xold_ref, w1_ref, b1_ref, w2_ref, b2_ref,
                pw1_ref, pb1_ref, pw2_ref, pb2_ref,
                xnew_ref, pcol_ref):
    agg = agg_ref[0] + agg_ref[1]
    h = jnp.maximum(
        jnp.dot(agg, w1_ref[...], preferred_element_type=_f32) + b1_ref[...],
        0.0)
    h2 = jnp.dot(h, w2_ref[...], preferred_element_type=_f32) + b2_ref[...]
    xnew_ref[...] = (jnp.maximum(h2, 0.0) + xold_ref[...]) * 0.5
    t = jnp.maximum(
        jnp.dot(h2, pw1_ref[...], preferred_element_type=_f32)
        + pb1_ref[...], 0.0)
    pcol_ref[...] = jnp.dot(t, pw2_ref[...],
                            preferred_element_type=_f32) + pb2_ref[...]


def _layer(agg2, xold, p, pp):
    return pl.pallas_call(
        _layer_body,
        grid=(GRID,),
        in_specs=[pl.BlockSpec((2, R, D), lambda i: (0, i, 0)),
                  pl.BlockSpec((R, D), lambda i: (i, 0)),
                  pl.BlockSpec((D, D), lambda i: (0, 0)),
                  pl.BlockSpec((1, D), lambda i: (0, 0)),
                  pl.BlockSpec((D, D), lambda i: (0, 0)),
                  pl.BlockSpec((1, D), lambda i: (0, 0)),
                  pl.BlockSpec((D, D), lambda i: (0, 0)),
                  pl.BlockSpec((1, D), lambda i: (0, 0)),
                  pl.BlockSpec((D, 1), lambda i: (0, 0)),
                  pl.BlockSpec((1, 1), lambda i: (0, 0))],
        out_specs=[pl.BlockSpec((R, D), lambda i: (i, 0)),
                   pl.BlockSpec((R, 1), lambda i: (i, 0))],
        out_shape=[jax.ShapeDtypeStruct((N, D), _f32),
                   jax.ShapeDtypeStruct((N, 1), _f32)],
    )(agg2, xold, p['W1'], p['b1'].reshape(1, D),
      p['W2'], p['b2'].reshape(1, D),
      pp['W1'], pp['b1'].reshape(1, D),
      pp['W2'], pp['b2'].reshape(1, 1))


def kernel(x_cons, x_vals, edge_index_c2v, edge_index_v2c,
           edge_attr_c2v, edge_attr_v2c, params):
    pr = params
    padE = NCHP * CK - E
    fill = jnp.arange(padE, dtype=_i32) % N

    def prep(ei, ea):
        src = jnp.concatenate([ei[0].astype(_i32), fill]).reshape(NCHP, CK)
        dst = jnp.concatenate([ei[1].astype(_i32), fill]).reshape(NCHP, CK)
        eap = jnp.concatenate(
            [ea.astype(_f32).reshape(E), jnp.zeros((padE,), _f32)]
        ).reshape(NCHP, CK)
        return jnp.stack([src, dst], axis=1), eap

    sd_c, ea_c = prep(edge_index_c2v, edge_attr_c2v)
    sd_v, ea_v = prep(edge_index_v2c, edge_attr_v2c)
    w_c, w_v = _weights(sd_c, ea_c, sd_v, ea_v)
    xc = _encode(x_cons, pr['enc_cons'])
    xv = _encode(x_vals, pr['enc_vals'])

    pv, pc = [], []
    for i in range(3):
        aggv = _spmm(xc, sd_c, w_c)
        aggc = _spmm(xv, sd_v, w_v)
        xv, pvi = _layer(aggv, xv, pr['convs'][i]['c2v'], pr['pred_vals'])
        xc, pci = _layer(aggc, xc, pr['convs'][i]['v2c'], pr['pred_cons'])
        pv.append(pvi)
        pc.append(pci)

    vals = jnp.concatenate(pv, axis=1)
    cons = jnp.concatenate(pc, axis=1)
    return (vals, cons)


# speedup vs baseline: 1.2171x; 1.0173x over previous
import functools

import jax
import jax.numpy as jnp
from jax import lax
from jax.experimental import pallas as pl
from jax.experimental.pallas import tpu as pltpu
from jax.experimental.pallas import tpu_sc as plsc

N = 25000
D = 64
E = 800000
CK = 128
NCH = E // CK
NCHP = 6400
SUP = 5
GRP = 10
WCH = NCHP // 32
NP = 25088
STRIPE = NP // 16
ROWS = 1560
R = 5000
GRID = N // R

_MESH = dict(core_axis_name="c", subcore_axis_name="s", num_cores=2,
             num_subcores=16)

_f32 = jnp.float32
_i32 = jnp.int32


def _fast_rsqrt(y):
    i = lax.bitcast_convert_type(y, _i32)
    i = jnp.int32(0x5F3759DF) - lax.shift_right_logical(i, 1)
    r = lax.bitcast_convert_type(i, _f32)
    for _ in range(3):
        r = r * (1.5 - 0.5 * y * r * r)
    return r


def _weights_body(sd_c, ea_c, sd_v, ea_v,
                  w_c, w_v,
                  deg_s_sh, deg_d_sh, rsq_s_sh, rsq_d_sh,
                  sdS, sdS2, eaS, eaS2, wS, wS2, valsA, valsB,
                  stripe, rsql_s, rsql_d, ssem, isemA, isemB, osem):
    c = lax.axis_index("c")
    s = lax.axis_index("s")

    def zero_stripe(v, _):
        stripe[pl.ds(v * 16, 16)] = jnp.zeros((16,), _f32)
        return 0
    lax.fori_loop(0, STRIPE // 16, zero_stripe, 0)
    pltpu.sync_copy(stripe, deg_s_sh.at[pl.ds(s * STRIPE, STRIPE)])
    pltpu.sync_copy(stripe, deg_d_sh.at[pl.ds(s * STRIPE, STRIPE)])
    plsc.subcore_barrier()

    per_tile = NCHP // 16
    start = s * per_tile
    npairs = per_tile // SUP // 2

    def set_vals(buf, base):
        v = lax.broadcast(jnp.where(base < NCH, 1.0, 0.0), (16,))
        for g in range(CK // 16):
            buf[pl.ds(g * 16, 16)] = v

    def deg_pass(sd):
        def pbody(pp, _):
            baseA = start + (2 * pp) * SUP
            baseB = baseA + SUP
            dA = pltpu.async_copy(sd.at[pl.ds(baseA, SUP)], sdS, isemA)
            dB = pltpu.async_copy(sd.at[pl.ds(baseB, SUP)], sdS2, isemB)
            set_vals(valsA, baseA)
            set_vals(valsB, baseB)
            descs = []
            dA.wait()
            for j in range(SUP):
                descs.append(pltpu.async_copy(
                    valsA, deg_s_sh.at[sdS.at[j, 0]], ssem, add=True))
                descs.append(pltpu.async_copy(
                    valsA, deg_d_sh.at[sdS.at[j, 1]], ssem, add=True))
            dB.wait()
            for j in range(SUP):
                descs.append(pltpu.async_copy(
                    valsB, deg_s_sh.at[sdS2.at[j, 0]], ssem, add=True))
                descs.append(pltpu.async_copy(
                    valsB, deg_d_sh.at[sdS2.at[j, 1]], ssem, add=True))
            for d in descs:
                d.wait()
            return 0
        lax.fori_loop(0, npairs, pbody, 0)

    @pl.when(c == 0)
    def _():
        deg_pass(sd_c)

    @pl.when(c == 1)
    def _():
        deg_pass(sd_v)

    plsc.subcore_barrier()

    def rsqrt_stripe(deg_sh, rsq_sh):
        pltpu.sync_copy(deg_sh.at[pl.ds(s * STRIPE, STRIPE)], stripe)

        def body(v, _):
            y = jnp.maximum(stripe[pl.ds(v * 16, 16)], 1.0)
            stripe[pl.ds(v * 16, 16)] = _fast_rsqrt(y)
            return 0
        lax.fori_loop(0, STRIPE // 16, body, 0)
        pltpu.sync_copy(stripe, rsq_sh.at[pl.ds(s * STRIPE, STRIPE)])

    rsqrt_stripe(deg_s_sh, rsq_s_sh)
    rsqrt_stripe(deg_d_sh, rsq_d_sh)
    plsc.subcore_barrier()

    pltpu.sync_copy(rsq_s_sh, rsql_s)
    pltpu.sync_copy(rsq_d_sh, rsql_d)

    def w_pass(sd, ea, wout):
        def compute_w(sdb, eab, wb):
            for j in range(SUP):
                for g in range(CK // 16):
                    si = sdb[j, 0, pl.ds(g * 16, 16)]
                    di = sdb[j, 1, pl.ds(g * 16, 16)]
                    gs = plsc.load_gather(rsql_s, [si])
                    gd = plsc.load_gather(rsql_d, [di])
                    wb[j, pl.ds(g * 16, 16)] = (
                        eab[j, pl.ds(g * 16, 16)] * gs * gd)

        def pbody(pp, _):
            baseA = start + (2 * pp) * SUP
            baseB = baseA + SUP
            dA = pltpu.async_copy(sd.at[pl.ds(baseA, SUP)], sdS, isemA)
            dAe = pltpu.async_copy(ea.at[pl.ds(baseA, SUP)], eaS, isemA)
            dB = pltpu.async_copy(sd.at[pl.ds(baseB, SUP)], sdS2, isemB)
            dBe = pltpu.async_copy(ea.at[pl.ds(baseB, SUP)], eaS2, isemB)
            dA.wait()
            dAe.wait()
            compute_w(sdS, eaS, wS)
            oA = pltpu.async_copy(wS, wout.at[pl.ds(baseA, SUP)], osem)
            dB.wait()
            dBe.wait()
            compute_w(sdS2, eaS2, wS2)
            oB = pltpu.async_copy(wS2, wout.at[pl.ds(baseB, SUP)], osem)
            oA.wait()
            oB.wait()
            return 0
        lax.fori_loop(0, npairs, pbody, 0)

    @pl.when(c == 0)
    def _():
        w_pass(sd_c, ea_c, w_c)

    @pl.when(c == 1)
    def _():
        w_pass(sd_v, ea_v, w_v)


def _weights(sd_c, ea_c, sd_v, ea_v):
    k = pl.kernel(
        _weights_body,
        out_type=[jax.ShapeDtypeStruct((NCHP, CK), _f32),
                  jax.ShapeDtypeStruct((NCHP, CK), _f32)],
        mesh=plsc.VectorSubcoreMesh(**_MESH),
        compiler_params=pltpu.CompilerParams(needs_layout_passes=False, use_tc_tiling_on_sc=False),
        scratch_types=[
            pltpu.VMEM_SHARED((NP,), _f32),
            pltpu.VMEM_SHARED((NP,), _f32),
            pltpu.VMEM_SHARED((NP,), _f32),
            pltpu.VMEM_SHARED((NP,), _f32),
            pltpu.VMEM((SUP, 2, CK), _i32),
            pltpu.VMEM((SUP, 2, CK), _i32),
            pltpu.VMEM((SUP, CK), _f32),
            pltpu.VMEM((SUP, CK), _f32),
            pltpu.VMEM((SUP, CK), _f32),
            pltpu.VMEM((SUP, CK), _f32),
            pltpu.VMEM((CK,), _f32),
            pltpu.VMEM((CK,), _f32),
            pltpu.VMEM((STRIPE,), _f32),
            pltpu.VMEM((NP,), _f32),
            pltpu.VMEM((NP,), _f32),
            pltpu.SemaphoreType.DMA,
            pltpu.SemaphoreType.DMA,
            pltpu.SemaphoreType.DMA,
            pltpu.SemaphoreType.DMA,
        ],
    )
    return k(sd_c, ea_c, sd_v, ea_v)


_ZFULL = ROWS // CK
_ZREM = ROWS - _ZFULL * CK
_TAIL = N - 16 * ROWS


def _spmm_body(x_hbm, sd_hbm, w_hbm,
               out_hbm,
               acc, sdA, sdB, wwA, wwB, r0, r1, r2,
               g0, g1, g2, s0, s1, s2, isemA, isemB):
    rows = (r0, r1, r2)
    gsem = (g0, g1, g2)
    ssem = (s0, s1, s2)
    c = lax.axis_index("c")
    s = lax.axis_index("s")

    @plsc.parallel_loop(0, CK)
    def _(rr):
        for m in range(D // 16):
            r0[rr, pl.ds(m * 16, 16)] = jnp.zeros((16,), _f32)

    def zero_acc(k, _):
        pltpu.sync_copy(r0, acc.at[pl.ds(s * ROWS + k * CK, CK)])
        return 0
    lax.fori_loop(0, _ZFULL, zero_acc, 0)
    pltpu.sync_copy(r0.at[pl.ds(0, _ZREM)],
                    acc.at[pl.ds(s * ROWS + _ZFULL * CK, _ZREM)])

    @pl.when(s == 0)
    def _():
        pltpu.sync_copy(r0.at[pl.ds(0, _TAIL)],
                        acc.at[pl.ds(16 * ROWS, _TAIL)])

    plsc.subcore_barrier()

    start = (c * 16 + s) * WCH

    def sref(k):
        return (sdA if k < SUP else sdB).at[k % SUP, 0]

    def dref(k):
        return (sdA if k < SUP else sdB).at[k % SUP, 1]

    def wref(k):
        return (wwA if k < SUP else wwB).at[k % SUP]

    def group(g, _):
        base = start + g * GRP
        dA = pltpu.async_copy(sd_hbm.at[pl.ds(base, SUP)], sdA, isemA)
        dAw = pltpu.async_copy(w_hbm.at[pl.ds(base, SUP)], wwA, isemA)
        dB = pltpu.async_copy(sd_hbm.at[pl.ds(base + SUP, SUP)], sdB, isemB)
        dBw = pltpu.async_copy(w_hbm.at[pl.ds(base + SUP, SUP)], wwB, isemB)
        dA.wait()
        dAw.wait()
        gd = {}
        sc = {}
        gd[0] = pltpu.async_copy(x_hbm.at[sref(0)], rows[0], gsem[0])
        gd[1] = pltpu.async_copy(x_hbm.at[sref(1)], rows[1], gsem[1])
        for k in range(2, GRP + 2):
            if k == SUP:
                dB.wait()
                dBw.wait()
            j = k - 2
            gd[j].wait()
            rr = rows[j % 3]
            wr = wref(j)

            @plsc.parallel_loop(0, CK, unroll=16)
            def _(e):
                wb = plsc.load_gather(wr, [lax.broadcast(e, (16,))])
                for m in range(D // 16):
                    rr[e, pl.ds(m * 16, 16)] = rr[e, pl.ds(m * 16, 16)] * wb

            sc[j] = pltpu.async_copy(rr, acc.at[dref(j)], ssem[j % 3],
                                     add=True)
            if k < GRP:
                if k >= 3:
                    sc[k - 3].wait()
                gd[k] = pltpu.async_copy(x_hbm.at[sref(k)], rows[k % 3],
                                         gsem[k % 3])
        for j in range(GRP - 3, GRP):
            sc[j].wait()
        return 0

    lax.fori_loop(0, WCH // GRP, group, 0)
    plsc.subcore_barrier()

    def out_copy(k, _):
        base = s * ROWS + k * CK
        pltpu.sync_copy(acc.at[pl.ds(base, CK)], r0)
        pltpu.sync_copy(r0, out_hbm.at[c, pl.ds(base, CK)])
        return 0
    lax.fori_loop(0, _ZFULL, out_copy, 0)
    rem_base = s * ROWS + _ZFULL * CK
    pltpu.sync_copy(acc.at[pl.ds(rem_base, _ZREM)], r0.at[pl.ds(0, _ZREM)])
    pltpu.sync_copy(r0.at[pl.ds(0, _ZREM)],
                    out_hbm.at[c, pl.ds(rem_base, _ZREM)])

    @pl.when(s == 0)
    def _():
        pltpu.sync_copy(acc.at[pl.ds(16 * ROWS, _TAIL)],
                        r0.at[pl.ds(0, _TAIL)])
        pltpu.sync_copy(r0.at[pl.ds(0, _TAIL)],
                        out_hbm.at[c, pl.ds(16 * ROWS, _TAIL)])


def _spmm(x, sd, w):
    k = pl.kernel(
        _spmm_body,
        out_type=jax.ShapeDtypeStruct((2, N, D), _f32),
        mesh=plsc.VectorSubcoreMesh(**_MESH),
        compiler_params=pltpu.CompilerParams(needs_layout_passes=False, use_tc_tiling_on_sc=False),
        scratch_types=(
            [pltpu.VMEM_SHARED((N, D), _f32)]
            + [pltpu.VMEM((SUP, 2, CK), _i32)] * 2
            + [pltpu.VMEM((SUP, CK), _f32)] * 2
            + [pltpu.VMEM((CK, D), _f32)] * 3
            + [pltpu.SemaphoreType.DMA] * 8
        ),
    )
    return k(x, sd, w)


def _enc1_body(x_ref, w1_ref, b1_ref, h_ref, st_ref):
    i = pl.program_id(0)
    h = jnp.dot(x_ref[...], w1_ref[...], preferred_element_type=_f32)
    h = h + b1_ref[...]
    h_ref[...] = h
    st = jnp.concatenate(
        [jnp.sum(h, axis=0, keepdims=True),
         jnp.sum(h * h, axis=0, keepdims=True),
         jnp.zeros((6, D), _f32)], axis=0)

    @pl.when(i == 0)
    def _():
        st_ref[...] = st

    @pl.when(i > 0)
    def _():
        st_ref[...] = st_ref[...] + st


def _enc2_body(h_ref, st_ref, g1_ref, be1_ref, w2_ref, b2_ref, o_ref):
    st = st_ref[...]
    mu = st[0:1] * (1.0 / N)
    var = st[1:2] * (1.0 / N) - mu * mu
    xn = (h_ref[...] - mu) * lax.rsqrt(var + 1e-5) * g1_ref[...] + be1_ref[...]
    xn = jnp.maximum(xn, 0.0)
    o_ref[...] = jnp.dot(xn, w2_ref[...],
                         preferred_element_type=_f32) + b2_ref[...]


def _encode(x, p):
    h, st = pl.pallas_call(
        _enc1_body,
        grid=(GRID,),
        in_specs=[pl.BlockSpec((R, D), lambda i: (i, 0)),
                  pl.BlockSpec((D, D), lambda i: (0, 0)),
                  pl.BlockSpec((1, D), lambda i: (0, 0))],
        out_specs=[pl.BlockSpec((R, D), lambda i: (i, 0)),
                   pl.BlockSpec((8, D), lambda i: (0, 0))],
        out_shape=[jax.ShapeDtypeStruct((N, D), _f32),
                   jax.ShapeDtypeStruct((8, D), _f32)],
    )(x, p['W1'], p['b1'].reshape(1, D))
    return pl.pallas_call(
        _enc2_body,
        grid=(GRID,),
        in_specs=[pl.BlockSpec((R, D), lambda i: (i, 0)),
                  pl.BlockSpec((8, D), lambda i: (0, 0)),
                  pl.BlockSpec((1, D), lambda i: (0, 0)),
                  pl.BlockSpec((1, D), lambda i: (0, 0)),
                  pl.BlockSpec((D, D), lambda i: (0, 0)),
                  pl.BlockSpec((1, D), lambda i: (0, 0))],
        out_specs=pl.BlockSpec((R, D), lambda i: (i, 0)),
        out_shape=jax.ShapeDtypeStruct((N, D), _f32),
    )(h, st, p['g1'].reshape(1, D), p['be1'].reshape(1, D),
      p['W2'], p['b2'].reshape(1, D))


def _layer_body(agg_ref, xold_ref, w1_ref, b1_ref, w2_ref, b2_ref,
                pw1_ref, pb1_ref, pw2_ref, pb2_ref,
                xnew_ref, pcol_ref):
    agg = agg_ref[0] + agg_ref[1]
    h = jnp.maximum(
        jnp.dot(agg, w1_ref[...], preferred_element_type=_f32) + b1_ref[...],
        0.0)
    h2 = jnp.dot(h, w2_ref[...], preferred_element_type=_f32) + b2_ref[...]
    xnew_ref[...] = (jnp.maximum(h2, 0.0) + xold_ref[...]) * 0.5
    t = jnp.maximum(
        jnp.dot(h2, pw1_ref[...], preferred_element_type=_f32)
        + pb1_ref[...], 0.0)
    pcol_ref[...] = jnp.dot(t, pw2_ref[...],
                            preferred_element_type=_f32) + pb2_ref[...]


def _layer(agg2, xold, p, pp):
    return pl.pallas_call(
        _layer_body,
        grid=(GRID,),
        in_specs=[pl.BlockSpec((2, R, D), lambda i: (0, i, 0)),
                  pl.BlockSpec((R, D), lambda i: (i, 0)),
                  pl.BlockSpec((D, D), lambda i: (0, 0)),
                  pl.BlockSpec((1, D), lambda i: (0, 0)),
                  pl.BlockSpec((D, D), lambda i: (0, 0)),
                  pl.BlockSpec((1, D), lambda i: (0, 0)),
                  pl.BlockSpec((D, D), lambda i: (0, 0)),
                  pl.BlockSpec((1, D), lambda i: (0, 0)),
                  pl.BlockSpec((D, 1), lambda i: (0, 0)),
                  pl.BlockSpec((1, 1), lambda i: (0, 0))],
        out_specs=[pl.BlockSpec((R, D), lambda i: (i, 0)),
                   pl.BlockSpec((R, 1), lambda i: (i, 0))],
        out_shape=[jax.ShapeDtypeStruct((N, D), _f32),
                   jax.ShapeDtypeStruct((N, 1), _f32)],
    )(agg2, xold, p['W1'], p['b1'].reshape(1, D),
      p['W2'], p['b2'].reshape(1, D),
      pp['W1'], pp['b1'].reshape(1, D),
      pp['W2'], pp['b2'].reshape(1, 1))


def kernel(x_cons, x_vals, edge_index_c2v, edge_index_v2c,
           edge_attr_c2v, edge_attr_v2c, params):
    pr = params
    padE = NCHP * CK - E
    fill = jnp.arange(padE, dtype=_i32) % N

    def prep(ei, ea):
        src = jnp.concatenate([ei[0].astype(_i32), fill]).reshape(NCHP, CK)
        dst = jnp.concatenate([ei[1].astype(_i32), fill]).reshape(NCHP, CK)
        eap = jnp.concatenate(
            [ea.astype(_f32).reshape(E), jnp.zeros((padE,), _f32)]
        ).reshape(NCHP, CK)
        return jnp.stack([src, dst], axis=1), eap

    sd_c, ea_c = prep(edge_index_c2v, edge_attr_c2v)
    sd_v, ea_v = prep(edge_index_v2c, edge_attr_v2c)
    w_c, w_v = _weights(sd_c, ea_c, sd_v, ea_v)
    xc = _encode(x_cons, pr['enc_cons'])
    xv = _encode(x_vals, pr['enc_vals'])

    pv, pc = [], []
    for i in range(3):
        aggv = _spmm(xc, sd_c, w_c)
        aggc = _spmm(xv, sd_v, w_v)
        xv, pvi = _layer(aggv, xv, pr['convs'][i]['c2v'], pr['pred_vals'])
        xc, pci = _layer(aggc, xc, pr['convs'][i]['v2c'], pr['pred_cons'])
        pv.append(pvi)
        pc.append(pci)

    vals = jnp.concatenate(pv, axis=1)
    cons = jnp.concatenate(pc, axis=1)
    return (vals, cons)


# speedup vs baseline: 1.2411x; 1.0197x over previous
import functools

import jax
import jax.numpy as jnp
from jax import lax
from jax.experimental import pallas as pl
from jax.experimental.pallas import tpu as pltpu
from jax.experimental.pallas import tpu_sc as plsc

N = 25000
D = 64
E = 800000
CK = 128
NCH = E // CK
NCHP = 6400
SUP = 5
GRP = 10
WCH = NCHP // 32
NP = 25088
STRIPE = NP // 16
ROWS = 1560
R = 5000
GRID = N // R

_MESH = dict(core_axis_name="c", subcore_axis_name="s", num_cores=2,
             num_subcores=16)

_f32 = jnp.float32
_i32 = jnp.int32


def _fast_rsqrt(y):
    i = lax.bitcast_convert_type(y, _i32)
    i = jnp.int32(0x5F3759DF) - lax.shift_right_logical(i, 1)
    r = lax.bitcast_convert_type(i, _f32)
    for _ in range(3):
        r = r * (1.5 - 0.5 * y * r * r)
    return r


def _weights_body(sd_c, ea_c, sd_v, ea_v,
                  w_c, w_v,
                  deg_s_sh, deg_d_sh, rsq_s_sh, rsq_d_sh,
                  sdS, sdS2, eaS, eaS2, wS, wS2, valsA, valsB,
                  stripe, rsql_s, rsql_d, ssem, isemA, isemB, osem):
    c = lax.axis_index("c")
    s = lax.axis_index("s")

    def zero_stripe(v, _):
        stripe[pl.ds(v * 16, 16)] = jnp.zeros((16,), _f32)
        return 0
    lax.fori_loop(0, STRIPE // 16, zero_stripe, 0)
    pltpu.sync_copy(stripe, deg_s_sh.at[pl.ds(s * STRIPE, STRIPE)])
    pltpu.sync_copy(stripe, deg_d_sh.at[pl.ds(s * STRIPE, STRIPE)])
    plsc.subcore_barrier()

    per_tile = NCHP // 16
    start = s * per_tile
    npairs = per_tile // SUP // 2

    def set_vals(buf, base):
        v = lax.broadcast(jnp.where(base < NCH, 1.0, 0.0), (16,))
        for g in range(CK // 16):
            buf[pl.ds(g * 16, 16)] = v

    def deg_pass(sd):
        def pbody(pp, _):
            baseA = start + (2 * pp) * SUP
            baseB = baseA + SUP
            dA = pltpu.async_copy(sd.at[pl.ds(baseA, SUP)], sdS, isemA)
            dB = pltpu.async_copy(sd.at[pl.ds(baseB, SUP)], sdS2, isemB)
            set_vals(valsA, baseA)
            set_vals(valsB, baseB)
            descs = []
            dA.wait()
            for j in range(SUP):
                descs.append(pltpu.async_copy(
                    valsA, deg_s_sh.at[sdS.at[j, 0]], ssem, add=True))
                descs.append(pltpu.async_copy(
                    valsA, deg_d_sh.at[sdS.at[j, 1]], ssem, add=True))
            dB.wait()
            for j in range(SUP):
                descs.append(pltpu.async_copy(
                    valsB, deg_s_sh.at[sdS2.at[j, 0]], ssem, add=True))
                descs.append(pltpu.async_copy(
                    valsB, deg_d_sh.at[sdS2.at[j, 1]], ssem, add=True))
            for d in descs:
                d.wait()
            return 0
        lax.fori_loop(0, npairs, pbody, 0)

    @pl.when(c == 0)
    def _():
        deg_pass(sd_c)

    @pl.when(c == 1)
    def _():
        deg_pass(sd_v)

    plsc.subcore_barrier()

    def rsqrt_stripe(deg_sh, rsq_sh):
        pltpu.sync_copy(deg_sh.at[pl.ds(s * STRIPE, STRIPE)], stripe)

        def body(v, _):
            y = jnp.maximum(stripe[pl.ds(v * 16, 16)], 1.0)
            stripe[pl.ds(v * 16, 16)] = _fast_rsqrt(y)
            return 0
        lax.fori_loop(0, STRIPE // 16, body, 0)
        pltpu.sync_copy(stripe, rsq_sh.at[pl.ds(s * STRIPE, STRIPE)])

    rsqrt_stripe(deg_s_sh, rsq_s_sh)
    rsqrt_stripe(deg_d_sh, rsq_d_sh)
    plsc.subcore_barrier()

    pltpu.sync_copy(rsq_s_sh, rsql_s)
    pltpu.sync_copy(rsq_d_sh, rsql_d)

    def w_pass(sd, ea, wout):
        def compute_w(sdb, eab, wb):
            for j in range(SUP):
                for g in range(CK // 16):
                    si = sdb[j, 0, pl.ds(g * 16, 16)]
                    di = sdb[j, 1, pl.ds(g * 16, 16)]
                    gs = plsc.load_gather(rsql_s, [si])
                    gd = plsc.load_gather(rsql_d, [di])
                    wb[j, pl.ds(g * 16, 16)] = (
                        eab[j, pl.ds(g * 16, 16)] * gs * gd)

        def pbody(pp, _):
            baseA = start + (2 * pp) * SUP
            baseB = baseA + SUP
            dA = pltpu.async_copy(sd.at[pl.ds(baseA, SUP)], sdS, isemA)
            dAe = pltpu.async_copy(ea.at[pl.ds(baseA, SUP)], eaS, isemA)
            dB = pltpu.async_copy(sd.at[pl.ds(baseB, SUP)], sdS2, isemB)
            dBe = pltpu.async_copy(ea.at[pl.ds(baseB, SUP)], eaS2, isemB)
            dA.wait()
            dAe.wait()
            compute_w(sdS, eaS, wS)
            oA = pltpu.async_copy(wS, wout.at[pl.ds(baseA, SUP)], osem)
            dB.wait()
            dBe.wait()
            compute_w(sdS2, eaS2, wS2)
            oB = pltpu.async_copy(wS2, wout.at[pl.ds(baseB, SUP)], osem)
            oA.wait()
            oB.wait()
            return 0
        lax.fori_loop(0, npairs, pbody, 0)

    @pl.when(c == 0)
    def _():
        w_pass(sd_c, ea_c, w_c)

    @pl.when(c == 1)
    def _():
        w_pass(sd_v, ea_v, w_v)


def _weights(sd_c, ea_c, sd_v, ea_v):
    k = pl.kernel(
        _weights_body,
        out_type=[jax.ShapeDtypeStruct((NCHP, CK), _f32),
                  jax.ShapeDtypeStruct((NCHP, CK), _f32)],
        mesh=plsc.VectorSubcoreMesh(**_MESH),
        compiler_params=pltpu.CompilerParams(needs_layout_passes=False, use_tc_tiling_on_sc=False),
        scratch_types=[
            pltpu.VMEM_SHARED((NP,), _f32),
            pltpu.VMEM_SHARED((NP,), _f32),
            pltpu.VMEM_SHARED((NP,), _f32),
            pltpu.VMEM_SHARED((NP,), _f32),
            pltpu.VMEM((SUP, 2, CK), _i32),
            pltpu.VMEM((SUP, 2, CK), _i32),
            pltpu.VMEM((SUP, CK), _f32),
            pltpu.VMEM((SUP, CK), _f32),
            pltpu.VMEM((SUP, CK), _f32),
            pltpu.VMEM((SUP, CK), _f32),
            pltpu.VMEM((CK,), _f32),
            pltpu.VMEM((CK,), _f32),
            pltpu.VMEM((STRIPE,), _f32),
            pltpu.VMEM((NP,), _f32),
            pltpu.VMEM((NP,), _f32),
            pltpu.SemaphoreType.DMA,
            pltpu.SemaphoreType.DMA,
            pltpu.SemaphoreType.DMA,
            pltpu.SemaphoreType.DMA,
        ],
    )
    return k(sd_c, ea_c, sd_v, ea_v)


_ZFULL = ROWS // CK
_ZREM = ROWS - _ZFULL * CK
_TAIL = N - 16 * ROWS


def _spmm_body(x_hbm, sd_hbm, w_hbm,
               out_hbm,
               acc, sdA, sdB, wwA, wwB, r0, r1, r2,
               g0, g1, g2, s0, s1, s2, isemA, isemB):
    rows = (r0, r1, r2)
    gsem = (g0, g1, g2)
    ssem = (s0, s1, s2)
    c = lax.axis_index("c")
    s = lax.axis_index("s")

    @plsc.parallel_loop(0, CK)
    def _(rr):
        for m in range(D // 16):
            r0[rr, pl.ds(m * 16, 16)] = jnp.zeros((16,), _f32)

    zdescs = [pltpu.async_copy(r0, acc.at[pl.ds(s * ROWS + k * CK, CK)],
                               isemA) for k in range(_ZFULL)]
    zdescs.append(pltpu.async_copy(
        r0.at[pl.ds(0, _ZREM)],
        acc.at[pl.ds(s * ROWS + _ZFULL * CK, _ZREM)], isemA))
    for d in zdescs:
        d.wait()

    @pl.when(s == 0)
    def _():
        pltpu.sync_copy(r0.at[pl.ds(0, _TAIL)],
                        acc.at[pl.ds(16 * ROWS, _TAIL)])

    plsc.subcore_barrier()

    start = (c * 16 + s) * WCH

    def sref(k):
        return (sdA if k < SUP else sdB).at[k % SUP, 0]

    def dref(k):
        return (sdA if k < SUP else sdB).at[k % SUP, 1]

    def wref(k):
        return (wwA if k < SUP else wwB).at[k % SUP]

    def group(g, _):
        base = start + g * GRP
        dA = pltpu.async_copy(sd_hbm.at[pl.ds(base, SUP)], sdA, isemA)
        dAw = pltpu.async_copy(w_hbm.at[pl.ds(base, SUP)], wwA, isemA)
        dB = pltpu.async_copy(sd_hbm.at[pl.ds(base + SUP, SUP)], sdB, isemB)
        dBw = pltpu.async_copy(w_hbm.at[pl.ds(base + SUP, SUP)], wwB, isemB)
        dA.wait()
        dAw.wait()
        gd = {}
        sc = {}
        gd[0] = pltpu.async_copy(x_hbm.at[sref(0)], rows[0], gsem[0])
        gd[1] = pltpu.async_copy(x_hbm.at[sref(1)], rows[1], gsem[1])
        for k in range(2, GRP + 2):
            if k == SUP:
                dB.wait()
                dBw.wait()
            j = k - 2
            gd[j].wait()
            rr = rows[j % 3]
            wr = wref(j)

            @plsc.parallel_loop(0, CK, unroll=16)
            def _(e):
                wb = plsc.load_gather(wr, [lax.broadcast(e, (16,))])
                for m in range(D // 16):
                    rr[e, pl.ds(m * 16, 16)] = rr[e, pl.ds(m * 16, 16)] * wb

            sc[j] = pltpu.async_copy(rr, acc.at[dref(j)], ssem[j % 3],
                                     add=True)
            if k < GRP:
                if k >= 3:
                    sc[k - 3].wait()
                gd[k] = pltpu.async_copy(x_hbm.at[sref(k)], rows[k % 3],
                                         gsem[k % 3])
        for j in range(GRP - 3, GRP):
            sc[j].wait()
        return 0

    lax.fori_loop(0, WCH // GRP, group, 0)
    plsc.subcore_barrier()

    din = {}
    dout = {}
    for k in range(3):
        din[k] = pltpu.async_copy(acc.at[pl.ds(s * ROWS + k * CK, CK)],
                                  rows[k], gsem[k])
    for k in range(_ZFULL):
        b = k % 3
        base = s * ROWS + k * CK
        din[k].wait()
        dout[k] = pltpu.async_copy(rows[b], out_hbm.at[c, pl.ds(base, CK)],
                                   ssem[b])
        if k + 3 < _ZFULL:
            dout[k].wait()
            din[k + 3] = pltpu.async_copy(
                acc.at[pl.ds(s * ROWS + (k + 3) * CK, CK)], rows[b],
                gsem[b])
    for k in range(_ZFULL - 3, _ZFULL):
        dout[k].wait()
    rem_base = s * ROWS + _ZFULL * CK
    pltpu.sync_copy(acc.at[pl.ds(rem_base, _ZREM)], r0.at[pl.ds(0, _ZREM)])
    pltpu.sync_copy(r0.at[pl.ds(0, _ZREM)],
                    out_hbm.at[c, pl.ds(rem_base, _ZREM)])

    @pl.when(s == 0)
    def _():
        pltpu.sync_copy(acc.at[pl.ds(16 * ROWS, _TAIL)],
                        r0.at[pl.ds(0, _TAIL)])
        pltpu.sync_copy(r0.at[pl.ds(0, _TAIL)],
                        out_hbm.at[c, pl.ds(16 * ROWS, _TAIL)])


def _spmm(x, sd, w):
    k = pl.kernel(
        _spmm_body,
        out_type=jax.ShapeDtypeStruct((2, N, D), _f32),
        mesh=plsc.VectorSubcoreMesh(**_MESH),
        compiler_params=pltpu.CompilerParams(needs_layout_passes=False, use_tc_tiling_on_sc=False),
        scratch_types=(
            [pltpu.VMEM_SHARED((N, D), _f32)]
            + [pltpu.VMEM((SUP, 2, CK), _i32)] * 2
            + [pltpu.VMEM((SUP, CK), _f32)] * 2
            + [pltpu.VMEM((CK, D), _f32)] * 3
            + [pltpu.SemaphoreType.DMA] * 8
        ),
    )
    return k(x, sd, w)


def _enc1_body(x_ref, w1_ref, b1_ref, h_ref, st_ref):
    i = pl.program_id(0)
    h = jnp.dot(x_ref[...], w1_ref[...], preferred_element_type=_f32)
    h = h + b1_ref[...]
    h_ref[...] = h
    st = jnp.concatenate(
        [jnp.sum(h, axis=0, keepdims=True),
         jnp.sum(h * h, axis=0, keepdims=True),
         jnp.zeros((6, D), _f32)], axis=0)

    @pl.when(i == 0)
    def _():
        st_ref[...] = st

    @pl.when(i > 0)
    def _():
        st_ref[...] = st_ref[...] + st


def _enc2_body(h_ref, st_ref, g1_ref, be1_ref, w2_ref, b2_ref, o_ref):
    st = st_ref[...]
    mu = st[0:1] * (1.0 / N)
    var = st[1:2] * (1.0 / N) - mu * mu
    xn = (h_ref[...] - mu) * lax.rsqrt(var + 1e-5) * g1_ref[...] + be1_ref[...]
    xn = jnp.maximum(xn, 0.0)
    o_ref[...] = jnp.dot(xn, w2_ref[...],
                         preferred_element_type=_f32) + b2_ref[...]


def _encode(x, p):
    h, st = pl.pallas_call(
        _enc1_body,
        grid=(GRID,),
        in_specs=[pl.BlockSpec((R, D), lambda i: (i, 0)),
                  pl.BlockSpec((D, D), lambda i: (0, 0)),
                  pl.BlockSpec((1, D), lambda i: (0, 0))],
        out_specs=[pl.BlockSpec((R, D), lambda i: (i, 0)),
                   pl.BlockSpec((8, D), lambda i: (0, 0))],
        out_shape=[jax.ShapeDtypeStruct((N, D), _f32),
                   jax.ShapeDtypeStruct((8, D), _f32)],
    )(x, p['W1'], p['b1'].reshape(1, D))
    return pl.pallas_call(
        _enc2_body,
        grid=(GRID,),
        in_specs=[pl.BlockSpec((R, D), lambda i: (i, 0)),
                  pl.BlockSpec((8, D), lambda i: (0, 0)),
                  pl.BlockSpec((1, D), lambda i: (0, 0)),
                  pl.BlockSpec((1, D), lambda i: (0, 0)),
                  pl.BlockSpec((D, D), lambda i: (0, 0)),
                  pl.BlockSpec((1, D), lambda i: (0, 0))],
        out_specs=pl.BlockSpec((R, D), lambda i: (i, 0)),
        out_shape=jax.ShapeDtypeStruct((N, D), _f32),
    )(h, st, p['g1'].reshape(1, D), p['be1'].reshape(1, D),
      p['W2'], p['b2'].reshape(1, D))


def _layer_body(agg_ref, xold_ref, w1_ref, b1_ref, w2_ref, b2_ref,
                pw1_ref, pb1_ref, pw2_ref, pb2_ref,
                xnew_ref, pcol_ref):
    agg = agg_ref[0] + agg_ref[1]
    h = jnp.maximum(
        jnp.dot(agg, w1_ref[...], preferred_element_type=_f32) + b1_ref[...],
        0.0)
    h2 = jnp.dot(h, w2_ref[...], preferred_element_type=_f32) + b2_ref[...]
    xnew_ref[...] = (jnp.maximum(h2, 0.0) + xold_ref[...]) * 0.5
    t = jnp.maximum(
        jnp.dot(h2, pw1_ref[...], preferred_element_type=_f32)
        + pb1_ref[...], 0.0)
    pcol_ref[...] = jnp.dot(t, pw2_ref[...],
                            preferred_element_type=_f32) + pb2_ref[...]


def _layer(agg2, xold, p, pp):
    return pl.pallas_call(
        _layer_body,
        grid=(GRID,),
        in_specs=[pl.BlockSpec((2, R, D), lambda i: (0, i, 0)),
                  pl.BlockSpec((R, D), lambda i: (i, 0)),
                  pl.BlockSpec((D, D), lambda i: (0, 0)),
                  pl.BlockSpec((1, D), lambda i: (0, 0)),
                  pl.BlockSpec((D, D), lambda i: (0, 0)),
                  pl.BlockSpec((1, D), lambda i: (0, 0)),
                  pl.BlockSpec((D, D), lambda i: (0, 0)),
                  pl.BlockSpec((1, D), lambda i: (0, 0)),
                  pl.BlockSpec((D, 1), lambda i: (0, 0)),
                  pl.BlockSpec((1, 1), lambda i: (0, 0))],
        out_specs=[pl.BlockSpec((R, D), lambda i: (i, 0)),
                   pl.BlockSpec((R, 1), lambda i: (i, 0))],
        out_shape=[jax.ShapeDtypeStruct((N, D), _f32),
                   jax.ShapeDtypeStruct((N, 1), _f32)],
    )(agg2, xold, p['W1'], p['b1'].reshape(1, D),
      p['W2'], p['b2'].reshape(1, D),
      pp['W1'], pp['b1'].reshape(1, D),
      pp['W2'], pp['b2'].reshape(1, 1))


def kernel(x_cons, x_vals, edge_index_c2v, edge_index_v2c,
           edge_attr_c2v, edge_attr_v2c, params):
    pr = params
    padE = NCHP * CK - E
    fill = jnp.arange(padE, dtype=_i32) % N

    def prep(ei, ea):
        src = jnp.concatenate([ei[0].astype(_i32), fill]).reshape(NCHP, CK)
        dst = jnp.concatenate([ei[1].astype(_i32), fill]).reshape(NCHP, CK)
        eap = jnp.concatenate(
            [ea.astype(_f32).reshape(E), jnp.zeros((padE,), _f32)]
        ).reshape(NCHP, CK)
        return jnp.stack([src, dst], axis=1), eap

    sd_c, ea_c = prep(edge_index_c2v, edge_attr_c2v)
    sd_v, ea_v = prep(edge_index_v2c, edge_attr_v2c)
    w_c, w_v = _weights(sd_c, ea_c, sd_v, ea_v)
    xc = _encode(x_cons, pr['enc_cons'])
    xv = _encode(x_vals, pr['enc_vals'])

    pv, pc = [], []
    for i in range(3):
        aggv = _spmm(xc, sd_c, w_c)
        aggc = _spmm(xv, sd_v, w_v)
        xv, pvi = _layer(aggv, xv, pr['convs'][i]['c2v'], pr['pred_vals'])
        xc, pci = _layer(aggc, xc, pr['convs'][i]['v2c'], pr['pred_cons'])
        pv.append(pvi)
        pc.append(pci)

    vals = jnp.concatenate(pv, axis=1)
    cons = jnp.concatenate(pc, axis=1)
    return (vals, cons)
